# baseline (speedup 1.0000x reference)
import functools

import jax
import jax.numpy as jnp
from jax import lax
from jax.experimental import pallas as pl
from jax.experimental.pallas import tpu as pltpu
from jax.experimental.pallas import tpu_sc as plsc

_BATCH = 16384
_L = 16
_N = 100000

_info = plsc.get_sparse_core_info()
_NC = _info.num_cores
_NS = _info.num_subcores
_NW = _NC * _NS
_BPW = _BATCH // _NW


def _body(f_hbm, cx_hbm, cy_hbm, k1_hbm, k2_hbm, idx_hbm, tbl_hbm,
          of_hbm, ocx_hbm, ocy_hbm, ok1_hbm, ok2_hbm,
          idx_v, gidx_v, g_v,
          f_v, cx_v, cy_v, k1_v, k2_v,
          of_v, ocx_v, ocy_v, ok1_v, ok2_v, sem, sem2):
  wid = lax.axis_index("s") * _NC + lax.axis_index("c")
  base = wid * _BPW
  sl_w = pl.ds(base, _BPW)

  cps = [pltpu.async_copy(src.at[sl_w], dst, sem2)
         for src, dst in ((f_hbm, f_v), (cx_hbm, cx_v), (cy_hbm, cy_v),
                          (k1_hbm, k1_v), (k2_hbm, k2_v))]

  pltpu.sync_copy(idx_hbm.at[sl_w], idx_v)

  for p in range(5):
    off = p * _N
    for j in range(_BPW // _L):
      gidx_v[pl.ds(p * _BPW + j * _L, _L)] = idx_v[pl.ds(j * _L, _L)] + off

  cpg = pltpu.async_copy(tbl_hbm.at[gidx_v], g_v, sem)

  for cp in cps:
    cp.wait()
  cpg.wait()

  for j in range(_BPW // _L):
    s = pl.ds(j * _L, _L)
    of_v[s] = f_v[s] * jnp.exp(g_v[pl.ds(j * _L, _L)])
    ocx_v[s] = cx_v[s] + g_v[pl.ds(_BPW + j * _L, _L)]
    ocy_v[s] = cy_v[s] + g_v[pl.ds(2 * _BPW + j * _L, _L)]
    ok1_v[s] = k1_v[s] + g_v[pl.ds(3 * _BPW + j * _L, _L)]
    ok2_v[s] = k2_v[s] + g_v[pl.ds(4 * _BPW + j * _L, _L)]

  cpo = [pltpu.async_copy(src, dst.at[sl_w], sem2)
         for src, dst in ((of_v, of_hbm), (ocx_v, ocx_hbm), (ocy_v, ocy_hbm),
                          (ok1_v, ok1_hbm), (ok2_v, ok2_hbm))]
  for cp in cpo:
    cp.wait()


_out = jax.ShapeDtypeStruct((_BATCH,), jnp.float32)
_f32v = pltpu.VMEM((_BPW,), jnp.float32)

_sc_call = functools.partial(
    pl.kernel,
    mesh=plsc.VectorSubcoreMesh(core_axis_name="c", subcore_axis_name="s"),
    out_type=(_out, _out, _out, _out, _out),
    scratch_types=[
        pltpu.VMEM((_BPW,), jnp.int32),
        pltpu.VMEM((5 * _BPW,), jnp.int32),
        pltpu.VMEM((5 * _BPW,), jnp.float32),
        _f32v, _f32v, _f32v, _f32v, _f32v,
        _f32v, _f32v, _f32v, _f32v, _f32v,
        pltpu.SemaphoreType.DMA,
        pltpu.SemaphoreType.DMA,
    ],
)(_body)


@jax.jit
def kernel(f, cx, cy, k1, k2, idx,
           focal_refinements, principal_point_refinements,
           distortion_refinements):
  idx = idx.astype(jnp.int32)
  tbl = jnp.concatenate([
      focal_refinements,
      principal_point_refinements[:, 0],
      principal_point_refinements[:, 1],
      distortion_refinements[:, 0],
      distortion_refinements[:, 1],
  ])
  return _sc_call(f, cx, cy, k1, k2, idx, tbl)


# speedup vs baseline: 1.4978x; 1.4978x over previous
import functools

import jax
import jax.numpy as jnp
from jax import lax
from jax.experimental import pallas as pl
from jax.experimental.pallas import tpu as pltpu
from jax.experimental.pallas import tpu_sc as plsc

_BATCH = 16384
_L = 16

_info = plsc.get_sparse_core_info()
_NC = _info.num_cores
_NS = _info.num_subcores
_NW = _NC * _NS
_BPW = _BATCH // _NW


def _body(f_hbm, cx_hbm, cy_hbm, k1_hbm, k2_hbm, idx_hbm,
          fr_hbm, ppx_hbm, ppy_hbm, drx_hbm, dry_hbm,
          of_hbm, ocx_hbm, ocy_hbm, ok1_hbm, ok2_hbm,
          idx_v, fr_v, ppx_v, ppy_v, dx_v, dy_v,
          f_v, cx_v, cy_v, k1_v, k2_v,
          of_v, ocx_v, ocy_v, ok1_v, ok2_v, sem, sem2):
  wid = lax.axis_index("s") * _NC + lax.axis_index("c")
  base = wid * _BPW
  sl_w = pl.ds(base, _BPW)

  cps = [pltpu.async_copy(src.at[sl_w], dst, sem2)
         for src, dst in ((f_hbm, f_v), (cx_hbm, cx_v), (cy_hbm, cy_v),
                          (k1_hbm, k1_v), (k2_hbm, k2_v))]

  pltpu.sync_copy(idx_hbm.at[sl_w], idx_v)

  cpg = [pltpu.async_copy(tbl.at[idx_v], dst, sem)
         for tbl, dst in ((fr_hbm, fr_v), (ppx_hbm, ppx_v), (ppy_hbm, ppy_v),
                          (drx_hbm, dx_v), (dry_hbm, dy_v))]

  for cp in cps + cpg:
    cp.wait()

  def _step(j, carry):
    s = pl.ds(j * _L, _L)
    of_v[s] = f_v[s] * jnp.exp(fr_v[s])
    ocx_v[s] = cx_v[s] + ppx_v[s]
    ocy_v[s] = cy_v[s] + ppy_v[s]
    ok1_v[s] = k1_v[s] + dx_v[s]
    ok2_v[s] = k2_v[s] + dy_v[s]
    return carry

  lax.fori_loop(0, _BPW // _L, _step, 0, unroll=4)

  cpo = [pltpu.async_copy(src, dst.at[sl_w], sem2)
         for src, dst in ((of_v, of_hbm), (ocx_v, ocx_hbm), (ocy_v, ocy_hbm),
                          (ok1_v, ok1_hbm), (ok2_v, ok2_hbm))]
  for cp in cpo:
    cp.wait()


_out = jax.ShapeDtypeStruct((_BATCH,), jnp.float32)
_f32v = pltpu.VMEM((_BPW,), jnp.float32)
_i32v = pltpu.VMEM((_BPW,), jnp.int32)

_sc_call = functools.partial(
    pl.kernel,
    mesh=plsc.VectorSubcoreMesh(core_axis_name="c", subcore_axis_name="s"),
    out_type=(_out, _out, _out, _out, _out),
    scratch_types=[
        _i32v,
        _f32v, _f32v, _f32v, _f32v, _f32v,
        _f32v, _f32v, _f32v, _f32v, _f32v,
        _f32v, _f32v, _f32v, _f32v, _f32v,
        pltpu.SemaphoreType.DMA,
        pltpu.SemaphoreType.DMA,
    ],
)(_body)


@jax.jit
def kernel(f, cx, cy, k1, k2, idx,
           focal_refinements, principal_point_refinements,
           distortion_refinements):
  idx = idx.astype(jnp.int32)
  return _sc_call(f, cx, cy, k1, k2, idx,
                  focal_refinements,
                  principal_point_refinements[:, 0],
                  principal_point_refinements[:, 1],
                  distortion_refinements[:, 0],
                  distortion_refinements[:, 1])


# speedup vs baseline: 1.5157x; 1.0120x over previous
import functools

import jax
import jax.numpy as jnp
from jax import lax
from jax.experimental import pallas as pl
from jax.experimental.pallas import tpu as pltpu
from jax.experimental.pallas import tpu_sc as plsc

_BATCH = 16384
_L = 16

_info = plsc.get_sparse_core_info()
_NC = _info.num_cores
_NS = _info.num_subcores
_NW = _NC * _NS
_BPW = _BATCH // _NW


def _body(f_hbm, cx_hbm, cy_hbm, k1_hbm, k2_hbm, idx_hbm,
          fr_hbm, ppx_hbm, ppy_hbm, drx_hbm, dry_hbm,
          of_hbm, ocx_hbm, ocy_hbm, ok1_hbm, ok2_hbm,
          idx_v, fr_v, ppx_v, ppy_v, dx_v, dy_v,
          f_v, cx_v, cy_v, k1_v, k2_v,
          of_v, ocx_v, ocy_v, ok1_v, ok2_v, sem, sem2):
  wid = lax.axis_index("s") * _NC + lax.axis_index("c")
  base = wid * _BPW
  sl_w = pl.ds(base, _BPW)

  cps = [pltpu.async_copy(src.at[sl_w], dst, sem2)
         for src, dst in ((f_hbm, f_v), (cx_hbm, cx_v), (cy_hbm, cy_v),
                          (k1_hbm, k1_v), (k2_hbm, k2_v))]

  pltpu.sync_copy(idx_hbm.at[sl_w], idx_v)

  cpg = [pltpu.async_copy(tbl.at[idx_v], dst, sem)
         for tbl, dst in ((fr_hbm, fr_v), (ppx_hbm, ppx_v), (ppy_hbm, ppy_v),
                          (drx_hbm, dx_v), (dry_hbm, dy_v))]

  for cp in cps + cpg:
    cp.wait()

  def _step(j, carry):
    s = pl.ds(j * _L, _L)
    of_v[s] = f_v[s] * jnp.exp(fr_v[s])
    ocx_v[s] = cx_v[s] + ppx_v[s]
    ocy_v[s] = cy_v[s] + ppy_v[s]
    ok1_v[s] = k1_v[s] + dx_v[s]
    ok2_v[s] = k2_v[s] + dy_v[s]
    return carry

  lax.fori_loop(0, _BPW // _L, _step, 0, unroll=4)

  cpo = [pltpu.async_copy(src, dst.at[sl_w], sem2)
         for src, dst in ((of_v, of_hbm), (ocx_v, ocx_hbm), (ocy_v, ocy_hbm),
                          (ok1_v, ok1_hbm), (ok2_v, ok2_hbm))]
  for cp in cpo:
    cp.wait()


_out = jax.ShapeDtypeStruct((_BATCH,), jnp.float32)
_f32v = pltpu.VMEM((_BPW,), jnp.float32)
_i32v = pltpu.VMEM((_BPW,), jnp.int32)

_sc_call = functools.partial(
    pl.kernel,
    mesh=plsc.VectorSubcoreMesh(core_axis_name="c", subcore_axis_name="s"),
    out_type=(_out, _out, _out, _out, _out),
    scratch_types=[
        _i32v,
        _f32v, _f32v, _f32v, _f32v, _f32v,
        _f32v, _f32v, _f32v, _f32v, _f32v,
        _f32v, _f32v, _f32v, _f32v, _f32v,
        pltpu.SemaphoreType.DMA,
        pltpu.SemaphoreType.DMA,
    ],
)(_body)


@jax.jit
def kernel(f, cx, cy, k1, k2, idx,
           focal_refinements, principal_point_refinements,
           distortion_refinements):
  idx = idx.astype(jnp.int32)
  e0 = jnp.array([1.0, 0.0], jnp.float32)
  e1 = jnp.array([0.0, 1.0], jnp.float32)
  return _sc_call(f, cx, cy, k1, k2, idx,
                  focal_refinements,
                  (principal_point_refinements * e0).sum(1),
                  (principal_point_refinements * e1).sum(1),
                  (distortion_refinements * e0).sum(1),
                  (distortion_refinements * e1).sum(1))
